# baseline (speedup 1.0000x reference)
import functools

import jax
import jax.numpy as jnp
import numpy as np
from jax import lax
from jax.experimental import pallas as pl
from jax.experimental.pallas import tpu as pltpu
import jax.experimental.pallas.tpu_sc as plsc

_S = 10
_NC = 2
_NS = 16
_NW = _NC * _NS
_L = 16


def _tf2x32(k1, k2, x0, x1):
  x0 = x0.astype(np.uint32).copy()
  x1 = x1.astype(np.uint32).copy()
  rot = [np.uint32(r) for r in (13, 15, 26, 6, 17, 29, 16, 24)]
  ks = [np.uint32(k1), np.uint32(k2), np.uint32(k1 ^ k2 ^ 0x1BD11BDA)]

  def rotl(x, d):
    return ((x << d) | (x >> np.uint32(32 - d))).astype(np.uint32)

  with np.errstate(over="ignore"):
    x0 = (x0 + ks[0]).astype(np.uint32)
    x1 = (x1 + ks[1]).astype(np.uint32)
    rr = [rot[:4], rot[4:]]
    kidx = [(1, 2), (2, 0), (0, 1), (1, 2), (2, 0)]
    for i in range(5):
      for r in rr[i % 2]:
        x0 = (x0 + x1).astype(np.uint32)
        x1 = rotl(x1, r)
        x1 = (x0 ^ x1).astype(np.uint32)
      a, b = kidx[i]
      x0 = (x0 + ks[a]).astype(np.uint32)
      x1 = (x1 + ks[b] + np.uint32(i + 1)).astype(np.uint32)
  return x0, x1


def _np_split(key):
  b1, b2 = _tf2x32(key[0], key[1], np.zeros(2, np.uint32),
                   np.arange(2, dtype=np.uint32))
  return np.stack([b1, b2], axis=1)


def _np_randint(key, shape, maxval):
  size = int(np.prod(shape))
  k1, k2 = _np_split(key)

  def bits32(k):
    b1, b2 = _tf2x32(k[0], k[1], np.zeros(size, np.uint32),
                     np.arange(size, dtype=np.uint32))
    return (b1 ^ b2).astype(np.uint32)

  hi, lo = bits32(k1), bits32(k2)
  span = np.uint32(maxval)
  mult = np.uint32((int(2 ** 16 % int(span))) ** 2 % int(span))
  with np.errstate(over="ignore"):
    off = ((hi % span) * mult + (lo % span)) % span
  return off.astype(np.int32).reshape(shape)


def _rand_picks(B, max_pos, max_neg):
  key42 = np.array([0, 42], dtype=np.uint32)
  r1, r2 = _np_split(key42)
  return (_np_randint(r1, (B, _S), max_pos),
          _np_randint(r2, (B, _S), max_neg))


def _dyn_gather(x, idx):
  return lax.gather(
      x, idx[:, None],
      lax.GatherDimensionNumbers(
          offset_dims=(), collapsed_slice_dims=(0,), start_index_map=(0,)),
      (1,), mode=lax.GatherScatterMode.PROMISE_IN_BOUNDS)


def _sc_dots(nb, rand_flat_pos, rand_flat_neg, sp_flat, sn_flat, all_emb,
             batch_emb, max_pos, max_neg):
  B = nb.shape[0]
  N, D = all_emb.shape
  RW = B // _NW
  SR = RW * _S
  _CH = 128
  NCH = SR // _CH
  DK = D // _L
  GCH = 8 * _S
  NGC = 2 * SR // GCH

  mesh = plsc.VectorSubcoreMesh(
      core_axis_name="c", subcore_axis_name="s",
      num_cores=_NC, num_subcores=_NS)

  @functools.partial(
      pl.kernel,
      out_type=(jax.ShapeDtypeStruct((B * _S,), jnp.float32),
                jax.ShapeDtypeStruct((B * _S,), jnp.float32)),
      mesh=mesh,
      scratch_types=[
          pltpu.VMEM((B,), jnp.int32),
          pltpu.VMEM((N,), jnp.int32),
          pltpu.VMEM((RW,), jnp.int32),
          pltpu.VMEM((SR,), jnp.int32),
          pltpu.VMEM((SR,), jnp.int32),
          pltpu.VMEM((SR,), jnp.int32),
          pltpu.VMEM((SR,), jnp.int32),
          pltpu.VMEM((2 * SR,), jnp.int32),
          pltpu.VMEM((RW, D), jnp.float32),
          pltpu.VMEM((3, GCH, D), jnp.float32),
          pltpu.VMEM((2 * SR,), jnp.float32),
          pltpu.VMEM((GCH * _L,), jnp.float32),
          pltpu.SemaphoreType.DMA,
          pltpu.SemaphoreType.DMA,
          pltpu.SemaphoreType.DMA,
          pltpu.SemaphoreType.DMA,
          pltpu.SemaphoreType.DMA,
      ],
      compiler_params=pltpu.CompilerParams(needs_layout_passes=False),
  )
  def k(nb_hbm, rp_hbm, rn_hbm, spf_hbm, snf_hbm, emb_hbm, be_hbm,
        pos_out, neg_out,
        nbv, win, wv, ridx, nbase, rvp, rvn, sidx, bel,
        ebuf, dots, ptmp, sem, semA, semB, semD, semC):
    wid = lax.axis_index("s") * _NC + lax.axis_index("c")
    base = wid * RW

    be_cp = pltpu.async_copy(be_hbm.at[pl.ds(base, RW)], bel, semC)
    pltpu.sync_copy(nb_hbm, nbv)

    lanes = lax.iota(jnp.int32, _L)

    def win_body(p, carry):
      v = nbv[pl.ds(p * _L, _L)]
      ml = lanes
      for kk in range(1, _L):
        rot = lax.rem(lanes + kk, _L)
        vr = _dyn_gather(v, rot)
        ml = jnp.where(vr == v, jnp.maximum(ml, rot), ml)
      plsc.store_scatter(win, [v], p * _L + ml)
      return carry
    lax.fori_loop(0, B // _L, win_body, 0)

    for q in range(RW // _L):
      idxv = nbv[pl.ds(base + q * _L, _L)]
      wv[pl.ds(q * _L, _L)] = plsc.load_gather(win, [idxv])

    for v in range(SR // _L):
      fl = lanes + v * _L
      il = lax.div(fl, _S)
      sl = lax.rem(fl, _S)
      wl = plsc.load_gather(wv, [il])
      ridx[pl.ds(v * _L, _L)] = wl * _S + sl
      nbase[pl.ds(v * _L, _L)] = plsc.load_gather(nbv, [base + il])

    cps = []
    for c in range(NCH):
      s_ = pl.ds(c * _CH, _CH)
      cps.append(pltpu.async_copy(rp_hbm.at[ridx.at[s_]], rvp.at[s_], sem))
      cps.append(pltpu.async_copy(rn_hbm.at[ridx.at[s_]], rvn.at[s_], sem))
    for cp in cps:
      cp.wait()

    for v in range(SR // _L):
      s_ = pl.ds(v * _L, _L)
      nb_l = nbase[s_]
      ridx[s_] = nb_l * max_pos + rvp[s_]
      nbase[s_] = nb_l * max_neg + rvn[s_]
    cps = []
    for c in range(NCH):
      s_ = pl.ds(c * _CH, _CH)
      cps.append(pltpu.async_copy(
          spf_hbm.at[ridx.at[s_]], sidx.at[s_], sem))
      cps.append(pltpu.async_copy(
          snf_hbm.at[nbase.at[s_]], sidx.at[pl.ds(SR + c * _CH, _CH)], sem))
    for cp in cps:
      cp.wait()
    be_cp.wait()

    lanes16 = lanes * _L
    IPC = GCH // _S

    def process(c, b, semX):
      pltpu.make_async_copy(
          emb_hbm.at[sidx.at[pl.ds(c * GCH, GCH)]], ebuf.at[b], semX).wait()

      i0 = lax.rem(c, NGC // 2) * IPC

      for ii in range(IPC):
        il = i0 + ii
        accs = [None] * _S
        for kk in range(DK):
          ber = bel[il, pl.ds(kk * _L, _L)]
          for s in range(_S):
            t = ebuf[b, ii * _S + s, pl.ds(kk * _L, _L)] * ber
            accs[s] = t if accs[s] is None else accs[s] + t
        for s in range(_S):
          ptmp[pl.ds((ii * _S + s) * _L, _L)] = accs[s]

      def red_body(g, carry):
        tot = plsc.load_gather(ptmp, [lanes16 + g * (_L * _L)])
        for kk in range(1, _L):
          tot = tot + plsc.load_gather(ptmp, [lanes16 + (g * (_L * _L) + kk)])
        dots[pl.ds(c * GCH + g * _L, _L)] = tot
        return carry
      lax.fori_loop(0, GCH // _L, red_body, 0)

      @pl.when(c < NGC - 3)
      def _():
        pltpu.async_copy(
            emb_hbm.at[sidx.at[pl.ds((c + 3) * GCH, GCH)]], ebuf.at[b], semX)

    pltpu.async_copy(emb_hbm.at[sidx.at[pl.ds(0, GCH)]], ebuf.at[0], semA)
    pltpu.async_copy(emb_hbm.at[sidx.at[pl.ds(GCH, GCH)]], ebuf.at[1], semB)
    pltpu.async_copy(emb_hbm.at[sidx.at[pl.ds(2 * GCH, GCH)]], ebuf.at[2], semD)

    def chunk_body(c, carry):
      @pl.when(lax.rem(c, 3) == 0)
      def _():
        process(c, 0, semA)

      @pl.when(lax.rem(c, 3) == 1)
      def _():
        process(c, 1, semB)

      @pl.when(lax.rem(c, 3) == 2)
      def _():
        process(c, 2, semD)
      return carry
    lax.fori_loop(0, NGC, chunk_body, 0)

    pltpu.sync_copy(dots.at[pl.ds(0, SR)], pos_out.at[pl.ds(base * _S, SR)])
    pltpu.sync_copy(dots.at[pl.ds(SR, SR)], neg_out.at[pl.ds(base * _S, SR)])

  return k(nb, rand_flat_pos, rand_flat_neg, sp_flat, sn_flat, all_emb,
           batch_emb)


def _softplus(z):
  return jnp.maximum(z, 0.0) + jnp.log1p(jnp.exp(-jnp.abs(z)))


def _tc_loss(dots_p, dots_n):
  n = dots_p.shape[0]

  def body(dp_ref, dn_ref, out_ref):
    tot = jnp.sum(_softplus(-dp_ref[...]) + _softplus(dn_ref[...]))
    out_ref[0, 0] = tot

  out = pl.pallas_call(
      body,
      out_specs=pl.BlockSpec(memory_space=pltpu.SMEM),
      out_shape=jax.ShapeDtypeStruct((1, 1), jnp.float32),
  )(dots_p.reshape(n // 128, 128), dots_n.reshape(n // 128, 128))
  return out.reshape(1)


def kernel(node_batch, batch_emb, all_emb, sample_pos, sample_neg):
  B = node_batch.shape[0]
  max_pos = sample_pos.shape[1]
  max_neg = sample_neg.shape[1]

  rand_pos, rand_neg = _rand_picks(B, max_pos, max_neg)
  rp = rand_pos.reshape(-1)
  rn = rand_neg.reshape(-1)

  nb = node_batch.astype(jnp.int32)
  spf = sample_pos.astype(jnp.int32).reshape(-1)
  snf = sample_neg.astype(jnp.int32).reshape(-1)
  ae = all_emb.astype(jnp.float32)
  be = batch_emb.astype(jnp.float32)

  dp, dn = _sc_dots(nb, rp, rn, spf, snf, ae, be, max_pos, max_neg)
  return _tc_loss(dp, dn) / B


# speedup vs baseline: 1.6653x; 1.6653x over previous
import functools

import jax
import jax.numpy as jnp
import numpy as np
from jax import lax
from jax.experimental import pallas as pl
from jax.experimental.pallas import tpu as pltpu
import jax.experimental.pallas.tpu_sc as plsc

_S = 10
_NC = 2
_NS = 16
_NW = _NC * _NS
_L = 16


def _tf2x32(k1, k2, x0, x1):
  x0 = x0.astype(np.uint32).copy()
  x1 = x1.astype(np.uint32).copy()
  rot = [np.uint32(r) for r in (13, 15, 26, 6, 17, 29, 16, 24)]
  ks = [np.uint32(k1), np.uint32(k2), np.uint32(k1 ^ k2 ^ 0x1BD11BDA)]

  def rotl(x, d):
    return ((x << d) | (x >> np.uint32(32 - d))).astype(np.uint32)

  with np.errstate(over="ignore"):
    x0 = (x0 + ks[0]).astype(np.uint32)
    x1 = (x1 + ks[1]).astype(np.uint32)
    rr = [rot[:4], rot[4:]]
    kidx = [(1, 2), (2, 0), (0, 1), (1, 2), (2, 0)]
    for i in range(5):
      for r in rr[i % 2]:
        x0 = (x0 + x1).astype(np.uint32)
        x1 = rotl(x1, r)
        x1 = (x0 ^ x1).astype(np.uint32)
      a, b = kidx[i]
      x0 = (x0 + ks[a]).astype(np.uint32)
      x1 = (x1 + ks[b] + np.uint32(i + 1)).astype(np.uint32)
  return x0, x1


def _np_split(key):
  b1, b2 = _tf2x32(key[0], key[1], np.zeros(2, np.uint32),
                   np.arange(2, dtype=np.uint32))
  return np.stack([b1, b2], axis=1)


def _np_randint(key, shape, maxval):
  size = int(np.prod(shape))
  k1, k2 = _np_split(key)

  def bits32(k):
    b1, b2 = _tf2x32(k[0], k[1], np.zeros(size, np.uint32),
                     np.arange(size, dtype=np.uint32))
    return (b1 ^ b2).astype(np.uint32)

  hi, lo = bits32(k1), bits32(k2)
  span = np.uint32(maxval)
  mult = np.uint32((int(2 ** 16 % int(span))) ** 2 % int(span))
  with np.errstate(over="ignore"):
    off = ((hi % span) * mult + (lo % span)) % span
  return off.astype(np.int32).reshape(shape)


def _rand_picks(B, max_pos, max_neg):
  key42 = np.array([0, 42], dtype=np.uint32)
  r1, r2 = _np_split(key42)
  return (_np_randint(r1, (B, _S), max_pos),
          _np_randint(r2, (B, _S), max_neg))


def _dyn_gather(x, idx):
  return lax.gather(
      x, idx[:, None],
      lax.GatherDimensionNumbers(
          offset_dims=(), collapsed_slice_dims=(0,), start_index_map=(0,)),
      (1,), mode=lax.GatherScatterMode.PROMISE_IN_BOUNDS)


def _sc_dots(nb, rand_flat_pos, rand_flat_neg, sp_flat, sn_flat, all_emb,
             batch_emb, max_pos, max_neg):
  B = nb.shape[0]
  N, D = all_emb.shape
  RW = B // _NW
  SR = RW * _S
  _CH = 128
  NCH = SR // _CH
  DK = D // _L
  GCH = 8 * _S
  NGC = 2 * SR // GCH

  mesh = plsc.VectorSubcoreMesh(
      core_axis_name="c", subcore_axis_name="s",
      num_cores=_NC, num_subcores=_NS)

  @functools.partial(
      pl.kernel,
      out_type=(jax.ShapeDtypeStruct((B * _S,), jnp.float32),
                jax.ShapeDtypeStruct((B * _S,), jnp.float32)),
      mesh=mesh,
      scratch_types=[
          pltpu.VMEM((B,), jnp.int32),
          pltpu.VMEM((N,), jnp.int32),
          pltpu.VMEM((RW,), jnp.int32),
          pltpu.VMEM((SR,), jnp.int32),
          pltpu.VMEM((SR,), jnp.int32),
          pltpu.VMEM((SR,), jnp.int32),
          pltpu.VMEM((SR,), jnp.int32),
          pltpu.VMEM((2 * SR,), jnp.int32),
          pltpu.VMEM((RW, D), jnp.float32),
          pltpu.VMEM((3, GCH, D), jnp.float32),
          pltpu.VMEM((2 * SR,), jnp.float32),
          pltpu.VMEM((GCH * _L,), jnp.float32),
          pltpu.SemaphoreType.DMA,
          pltpu.SemaphoreType.DMA,
          pltpu.SemaphoreType.DMA,
          pltpu.SemaphoreType.DMA,
          pltpu.SemaphoreType.DMA,
      ],
      compiler_params=pltpu.CompilerParams(needs_layout_passes=False),
  )
  def k(nb_hbm, rp_hbm, rn_hbm, spf_hbm, snf_hbm, emb_hbm, be_hbm,
        pos_out, neg_out,
        nbv, win, wv, ridx, nbase, rvp, rvn, sidx, bel,
        ebuf, dots, ptmp, sem, semA, semB, semD, semC):
    wid = lax.axis_index("s") * _NC + lax.axis_index("c")
    base = wid * RW

    be_cp = pltpu.async_copy(be_hbm.at[pl.ds(base, RW)], bel, semC)
    pltpu.sync_copy(nb_hbm, nbv)

    lanes = lax.iota(jnp.int32, _L)

    def win_body(p, carry):
      v = nbv[pl.ds(p * _L, _L)]
      ml = lanes
      for kk in range(1, _L):
        rot = lax.rem(lanes + kk, _L)
        vr = _dyn_gather(v, rot)
        ml = jnp.where(vr == v, jnp.maximum(ml, rot), ml)
      plsc.store_scatter(win, [v], p * _L + ml)
      return carry
    lax.fori_loop(0, B // _L, win_body, 0)

    for q in range(RW // _L):
      idxv = nbv[pl.ds(base + q * _L, _L)]
      wv[pl.ds(q * _L, _L)] = plsc.load_gather(win, [idxv])

    for v in range(SR // _L):
      fl = lanes + v * _L
      il = lax.div(fl, _S)
      sl = lax.rem(fl, _S)
      wl = plsc.load_gather(wv, [il])
      ridx[pl.ds(v * _L, _L)] = wl * _S + sl
      nbase[pl.ds(v * _L, _L)] = plsc.load_gather(nbv, [base + il])

    cps = []
    for c in range(NCH):
      s_ = pl.ds(c * _CH, _CH)
      cps.append(pltpu.async_copy(rp_hbm.at[ridx.at[s_]], rvp.at[s_], sem))
      cps.append(pltpu.async_copy(rn_hbm.at[ridx.at[s_]], rvn.at[s_], sem))
    for cp in cps:
      cp.wait()

    for v in range(SR // _L):
      s_ = pl.ds(v * _L, _L)
      nb_l = nbase[s_]
      ridx[s_] = nb_l * max_pos + rvp[s_]
      nbase[s_] = nb_l * max_neg + rvn[s_]
    cps = []
    for c in range(NCH):
      s_ = pl.ds(c * _CH, _CH)
      cps.append(pltpu.async_copy(
          spf_hbm.at[ridx.at[s_]], sidx.at[s_], sem))
      cps.append(pltpu.async_copy(
          snf_hbm.at[nbase.at[s_]], sidx.at[pl.ds(SR + c * _CH, _CH)], sem))
    for cp in cps:
      cp.wait()
    be_cp.wait()

    lanes16 = lanes * _L
    IPC = GCH // _S

    def process(c, b, semX):
      pltpu.make_async_copy(
          emb_hbm.at[sidx.at[pl.ds(c * GCH, GCH)]], ebuf.at[b], semX).wait()

      i0 = lax.rem(c, NGC // 2) * IPC

      def row_body(ii, carry):
        il = i0 + ii
        accs = [None] * _S
        for kk in range(DK):
          ber = bel[il, pl.ds(kk * _L, _L)]
          for s in range(_S):
            t = ebuf[b, ii * _S + s, pl.ds(kk * _L, _L)] * ber
            accs[s] = t if accs[s] is None else accs[s] + t
        for s in range(_S):
          ptmp[pl.ds((ii * _S + s) * _L, _L)] = accs[s]
        return carry
      lax.fori_loop(0, IPC, row_body, 0)

      def red_body(g, carry):
        tot = plsc.load_gather(ptmp, [lanes16 + g * (_L * _L)])
        for kk in range(1, _L):
          tot = tot + plsc.load_gather(ptmp, [lanes16 + (g * (_L * _L) + kk)])
        dots[pl.ds(c * GCH + g * _L, _L)] = tot
        return carry
      lax.fori_loop(0, GCH // _L, red_body, 0)

      @pl.when(c < NGC - 3)
      def _():
        pltpu.async_copy(
            emb_hbm.at[sidx.at[pl.ds((c + 3) * GCH, GCH)]], ebuf.at[b], semX)

    pltpu.async_copy(emb_hbm.at[sidx.at[pl.ds(0, GCH)]], ebuf.at[0], semA)
    pltpu.async_copy(emb_hbm.at[sidx.at[pl.ds(GCH, GCH)]], ebuf.at[1], semB)
    pltpu.async_copy(emb_hbm.at[sidx.at[pl.ds(2 * GCH, GCH)]], ebuf.at[2], semD)

    def chunk_body(c, carry):
      @pl.when(lax.rem(c, 3) == 0)
      def _():
        process(c, 0, semA)

      @pl.when(lax.rem(c, 3) == 1)
      def _():
        process(c, 1, semB)

      @pl.when(lax.rem(c, 3) == 2)
      def _():
        process(c, 2, semD)
      return carry
    lax.fori_loop(0, NGC, chunk_body, 0)

    pltpu.sync_copy(dots.at[pl.ds(0, SR)], pos_out.at[pl.ds(base * _S, SR)])
    pltpu.sync_copy(dots.at[pl.ds(SR, SR)], neg_out.at[pl.ds(base * _S, SR)])

  return k(nb, rand_flat_pos, rand_flat_neg, sp_flat, sn_flat, all_emb,
           batch_emb)


def _softplus(z):
  return jnp.maximum(z, 0.0) + jnp.log1p(jnp.exp(-jnp.abs(z)))


def _tc_loss(dots_p, dots_n):
  n = dots_p.shape[0]

  def body(dp_ref, dn_ref, out_ref):
    tot = jnp.sum(_softplus(-dp_ref[...]) + _softplus(dn_ref[...]))
    out_ref[0, 0] = tot

  out = pl.pallas_call(
      body,
      out_specs=pl.BlockSpec(memory_space=pltpu.SMEM),
      out_shape=jax.ShapeDtypeStruct((1, 1), jnp.float32),
  )(dots_p.reshape(n // 128, 128), dots_n.reshape(n // 128, 128))
  return out.reshape(1)


def kernel(node_batch, batch_emb, all_emb, sample_pos, sample_neg):
  B = node_batch.shape[0]
  max_pos = sample_pos.shape[1]
  max_neg = sample_neg.shape[1]

  rand_pos, rand_neg = _rand_picks(B, max_pos, max_neg)
  rp = rand_pos.reshape(-1)
  rn = rand_neg.reshape(-1)

  nb = node_batch.astype(jnp.int32)
  spf = sample_pos.astype(jnp.int32).reshape(-1)
  snf = sample_neg.astype(jnp.int32).reshape(-1)
  ae = all_emb.astype(jnp.float32)
  be = batch_emb.astype(jnp.float32)

  dp, dn = _sc_dots(nb, rp, rn, spf, snf, ae, be, max_pos, max_neg)
  return _tc_loss(dp, dn) / B
